# baseline (speedup 1.0000x reference)
import functools

import jax
import jax.numpy as jnp
from jax import lax
from jax.experimental import pallas as pl
from jax.experimental.pallas import tpu as pltpu
from jax.experimental.pallas import tpu_sc as plsc

D_MODEL = 2048
N_EXPERTS = 16
N_GROUPS = 4
GROUP_SIZE = 4
K = 2
N_TOK = 4 * 4096
NW = 32
N_CHUNK = 1
TOK_C = N_TOK // N_CHUNK
TPW = TOK_C // NW
LANES = 16
W_ROWS = 24
NEG = float(jnp.finfo(jnp.float32).min)


MM_BLK = 2048


def _mm_body(x_ref, w_ref, o_ref):
    out = lax.dot_general(
        w_ref[...], x_ref[...],
        dimension_numbers=(((1,), (1,)), ((), ())),
        preferred_element_type=jnp.float32,
    )
    for k in range(MM_BLK // TPW):
        o_ref[k] = out[:, k * TPW:(k + 1) * TPW]


def _logits_tc(x2d, w_cat):
    return pl.pallas_call(
        _mm_body,
        grid=(TOK_C // MM_BLK,),
        in_specs=[
            pl.BlockSpec((MM_BLK, D_MODEL), lambda i: (i, 0)),
            pl.BlockSpec((W_ROWS, D_MODEL), lambda i: (0, 0)),
        ],
        out_specs=pl.BlockSpec((MM_BLK // TPW, W_ROWS, TPW),
                               lambda i: (i, 0, 0)),
        out_shape=jax.ShapeDtypeStruct((NW, W_ROWS, TPW), jnp.float32),
    )(x2d, w_cat)


def _sc_route(lg_hbm, out_hbm, imp_hbm, cnt_hbm, lg_v, out_v, imp_a, cnt_a):
    wid = lax.axis_index("s") * 2 + lax.axis_index("c")
    pltpu.sync_copy(lg_hbm.at[wid], lg_v)

    zf = jnp.zeros((LANES,), jnp.float32)
    for j in range(N_EXPERTS):
        imp_a[j] = zf
        cnt_a[j] = zf

    def body(t, carry):
        off = t * LANES
        e = [lg_v[j, pl.ds(off, LANES)] for j in range(N_EXPERTS)]
        g = [lg_v[N_EXPERTS + j, pl.ds(off, LANES)] for j in range(N_GROUPS)]

        gmax = jnp.maximum(jnp.maximum(g[0], g[1]), jnp.maximum(g[2], g[3]))
        in_g = [g[j] == gmax for j in range(N_GROUPS)]
        gidx = jnp.full((LANES,), N_GROUPS - 1, jnp.int32)
        for j in range(N_GROUPS - 2, -1, -1):
            gidx = jnp.where(in_g[j], j, gidx)
        sel_g = [gidx == j for j in range(N_GROUPS)]

        a = [jnp.where(sel_g[0], e[s],
                       jnp.where(sel_g[1], e[GROUP_SIZE + s],
                                 jnp.where(sel_g[2], e[2 * GROUP_SIZE + s],
                                           e[3 * GROUP_SIZE + s])))
             for s in range(GROUP_SIZE)]

        m = jnp.maximum(jnp.maximum(a[0], a[1]), jnp.maximum(a[2], a[3]))
        ex = [jnp.exp(a[s] - m) for s in range(GROUP_SIZE)]
        inv = 1.0 / (ex[0] + ex[1] + ex[2] + ex[3])
        px = [ex[s] * inv for s in range(GROUP_SIZE)]

        top1 = [a[s] == m for s in range(GROUP_SIZE)]
        s1 = jnp.full((LANES,), GROUP_SIZE - 1, jnp.int32)
        for s in range(GROUP_SIZE - 2, -1, -1):
            s1 = jnp.where(top1[s], s, s1)
        m2 = jnp.full((LANES,), NEG, jnp.float32)
        ex1 = [s1 == s for s in range(GROUP_SIZE)]
        for s in range(GROUP_SIZE):
            m2 = jnp.maximum(m2, jnp.where(ex1[s], NEG, a[s]))
        top2 = [(a[s] == m2) & (s1 != s) for s in range(GROUP_SIZE)]
        s2 = jnp.zeros((LANES,), jnp.int32)
        for s in range(GROUP_SIZE - 1, -1, -1):
            s2 = jnp.where(top2[s], s, s2)
        base = gidx * GROUP_SIZE
        i1 = base + s1
        i2 = base + s2

        d = jnp.exp(m2 - m)
        w1 = 1.0 / (1.0 + d)
        w2 = d * w1

        one = jnp.ones((LANES,), jnp.float32)
        zero = jnp.zeros((LANES,), jnp.float32)
        c1 = [jnp.where(s1 == s, one, zero) + jnp.where(s2 == s, one, zero)
          for s in range(GROUP_SIZE)]
        for j in range(N_EXPERTS):
            gsel = sel_g[j // GROUP_SIZE]
            p = jnp.where(gsel, px[j % GROUP_SIZE], zero)
            out_v[j, pl.ds(off, LANES)] = p
            plsc.addupdate(imp_a.at[j], p)
            plsc.addupdate(cnt_a.at[j],
                           jnp.where(gsel, c1[j % GROUP_SIZE], zero))

        out_v[N_EXPERTS + 0, pl.ds(off, LANES)] = w1
        out_v[N_EXPERTS + 1, pl.ds(off, LANES)] = w2
        out_v[N_EXPERTS + 2, pl.ds(off, LANES)] = i1.astype(jnp.float32)
        out_v[N_EXPERTS + 3, pl.ds(off, LANES)] = i2.astype(jnp.float32)
        return carry

    lax.fori_loop(0, TPW // LANES, body, 0)

    pltpu.sync_copy(out_v, out_hbm.at[wid])
    pltpu.sync_copy(imp_a, imp_hbm.at[wid])
    pltpu.sync_copy(cnt_a, cnt_hbm.at[wid])


_sc_route_call = functools.partial(
    pl.kernel,
    out_type=(
        jax.ShapeDtypeStruct((NW, N_EXPERTS + 2 * K, TPW), jnp.float32),
        jax.ShapeDtypeStruct((NW, N_EXPERTS, LANES), jnp.float32),
        jax.ShapeDtypeStruct((NW, N_EXPERTS, LANES), jnp.float32),
    ),
    mesh=plsc.VectorSubcoreMesh(core_axis_name="c", subcore_axis_name="s"),
    scratch_types=[
        pltpu.VMEM((W_ROWS, TPW), jnp.float32),
        pltpu.VMEM((N_EXPERTS + 2 * K, TPW), jnp.float32),
        pltpu.VMEM((N_EXPERTS, LANES), jnp.float32),
        pltpu.VMEM((N_EXPERTS, LANES), jnp.float32),
    ],
)(_sc_route)


@jax.jit
def kernel(x, W_expert, W_group):
    x2d = x.reshape(N_TOK, D_MODEL)
    w_cat = jnp.concatenate(
        [W_expert, W_group,
         jnp.zeros((W_ROWS - N_EXPERTS - N_GROUPS, D_MODEL), jnp.float32)],
        axis=0)
    parts = []
    for c in range(N_CHUNK):
        logits = _logits_tc(
            lax.slice_in_dim(x2d, c * TOK_C, (c + 1) * TOK_C, axis=0), w_cat)
        parts.append(_sc_route_call(logits))

    packed = jnp.concatenate(
        [jnp.transpose(p[0], (0, 2, 1)).reshape(TOK_C, N_EXPERTS + 2 * K)
         for p in parts],
        axis=0)
    probs_full = packed[:, :N_EXPERTS].reshape(4, 4096, N_EXPERTS)
    scores = packed[:, N_EXPERTS:N_EXPERTS + K].reshape(4, 4096, K)
    idx = packed[:, N_EXPERTS + K:].astype(jnp.int32).reshape(4, 4096, K)
    imp_p = sum(jnp.sum(p[1], axis=(0, 2)) for p in parts)
    cnt_p = sum(jnp.sum(p[2], axis=(0, 2)) for p in parts)
    importance = imp_p / float(N_TOK)
    load = cnt_p / jnp.maximum(jnp.sum(cnt_p), 1.0)
    return (idx, scores, probs_full, importance, load)


# speedup vs baseline: 1.0320x; 1.0320x over previous
import functools

import jax
import jax.numpy as jnp
from jax import lax
from jax.experimental import pallas as pl
from jax.experimental.pallas import tpu as pltpu
from jax.experimental.pallas import tpu_sc as plsc

D_MODEL = 2048
N_EXPERTS = 16
N_GROUPS = 4
GROUP_SIZE = 4
K = 2
N_TOK = 4 * 4096
NW = 32
N_CHUNK = 1
TOK_C = N_TOK // N_CHUNK
TPW = TOK_C // NW
LANES = 16
W_ROWS = 24
NEG = float(jnp.finfo(jnp.float32).min)


MM_BLK = 1024


def _mm_body(x_ref, w_ref, o_ref):
    out = lax.dot_general(
        w_ref[...], x_ref[...],
        dimension_numbers=(((1,), (1,)), ((), ())),
        preferred_element_type=jnp.float32,
    )
    for k in range(MM_BLK // TPW):
        o_ref[k] = out[:, k * TPW:(k + 1) * TPW]


def _logits_tc(x2d, w_cat):
    return pl.pallas_call(
        _mm_body,
        grid=(TOK_C // MM_BLK,),
        in_specs=[
            pl.BlockSpec((MM_BLK, D_MODEL), lambda i: (i, 0)),
            pl.BlockSpec((W_ROWS, D_MODEL), lambda i: (0, 0)),
        ],
        out_specs=pl.BlockSpec((MM_BLK // TPW, W_ROWS, TPW),
                               lambda i: (i, 0, 0)),
        out_shape=jax.ShapeDtypeStruct((NW, W_ROWS, TPW), jnp.float32),
    )(x2d, w_cat)


def _sc_route(lg_hbm, out_hbm, imp_hbm, cnt_hbm, lg_v, out_v, imp_a, cnt_a):
    wid = lax.axis_index("s") * 2 + lax.axis_index("c")
    pltpu.sync_copy(lg_hbm.at[wid], lg_v)

    zf = jnp.zeros((LANES,), jnp.float32)
    for j in range(N_EXPERTS):
        imp_a[j] = zf
        cnt_a[j] = zf

    def body(t, carry):
        off = t * LANES
        e = [lg_v[j, pl.ds(off, LANES)] for j in range(N_EXPERTS)]
        g = [lg_v[N_EXPERTS + j, pl.ds(off, LANES)] for j in range(N_GROUPS)]

        gmax = jnp.maximum(jnp.maximum(g[0], g[1]), jnp.maximum(g[2], g[3]))
        in_g = [g[j] == gmax for j in range(N_GROUPS)]
        gidx = jnp.full((LANES,), N_GROUPS - 1, jnp.int32)
        for j in range(N_GROUPS - 2, -1, -1):
            gidx = jnp.where(in_g[j], j, gidx)
        sel_g = [gidx == j for j in range(N_GROUPS)]

        a = [jnp.where(sel_g[0], e[s],
                       jnp.where(sel_g[1], e[GROUP_SIZE + s],
                                 jnp.where(sel_g[2], e[2 * GROUP_SIZE + s],
                                           e[3 * GROUP_SIZE + s])))
             for s in range(GROUP_SIZE)]

        m = jnp.maximum(jnp.maximum(a[0], a[1]), jnp.maximum(a[2], a[3]))
        ex = [jnp.exp(a[s] - m) for s in range(GROUP_SIZE)]
        inv = 1.0 / (ex[0] + ex[1] + ex[2] + ex[3])
        px = [ex[s] * inv for s in range(GROUP_SIZE)]

        top1 = [a[s] == m for s in range(GROUP_SIZE)]
        s1 = jnp.full((LANES,), GROUP_SIZE - 1, jnp.int32)
        for s in range(GROUP_SIZE - 2, -1, -1):
            s1 = jnp.where(top1[s], s, s1)
        m2 = jnp.full((LANES,), NEG, jnp.float32)
        ex1 = [s1 == s for s in range(GROUP_SIZE)]
        for s in range(GROUP_SIZE):
            m2 = jnp.maximum(m2, jnp.where(ex1[s], NEG, a[s]))
        top2 = [(a[s] == m2) & (s1 != s) for s in range(GROUP_SIZE)]
        s2 = jnp.zeros((LANES,), jnp.int32)
        for s in range(GROUP_SIZE - 1, -1, -1):
            s2 = jnp.where(top2[s], s, s2)
        base = gidx * GROUP_SIZE
        i1 = base + s1
        i2 = base + s2

        d = jnp.exp(m2 - m)
        w1 = 1.0 / (1.0 + d)
        w2 = d * w1

        one = jnp.ones((LANES,), jnp.float32)
        zero = jnp.zeros((LANES,), jnp.float32)
        c1 = [jnp.where(s1 == s, one, zero) + jnp.where(s2 == s, one, zero)
          for s in range(GROUP_SIZE)]
        for j in range(N_EXPERTS):
            gsel = sel_g[j // GROUP_SIZE]
            p = jnp.where(gsel, px[j % GROUP_SIZE], zero)
            out_v[j, pl.ds(off, LANES)] = p
            plsc.addupdate(imp_a.at[j], p)
            plsc.addupdate(cnt_a.at[j],
                           jnp.where(gsel, c1[j % GROUP_SIZE], zero))

        out_v[N_EXPERTS + 0, pl.ds(off, LANES)] = w1
        out_v[N_EXPERTS + 1, pl.ds(off, LANES)] = w2
        out_v[N_EXPERTS + 2, pl.ds(off, LANES)] = i1.astype(jnp.float32)
        out_v[N_EXPERTS + 3, pl.ds(off, LANES)] = i2.astype(jnp.float32)
        return carry

    lax.fori_loop(0, TPW // LANES, body, 0)

    pltpu.sync_copy(out_v, out_hbm.at[wid])
    pltpu.sync_copy(imp_a, imp_hbm.at[wid])
    pltpu.sync_copy(cnt_a, cnt_hbm.at[wid])


_sc_route_call = functools.partial(
    pl.kernel,
    out_type=(
        jax.ShapeDtypeStruct((NW, N_EXPERTS + 2 * K, TPW), jnp.float32),
        jax.ShapeDtypeStruct((NW, N_EXPERTS, LANES), jnp.float32),
        jax.ShapeDtypeStruct((NW, N_EXPERTS, LANES), jnp.float32),
    ),
    mesh=plsc.VectorSubcoreMesh(core_axis_name="c", subcore_axis_name="s"),
    scratch_types=[
        pltpu.VMEM((W_ROWS, TPW), jnp.float32),
        pltpu.VMEM((N_EXPERTS + 2 * K, TPW), jnp.float32),
        pltpu.VMEM((N_EXPERTS, LANES), jnp.float32),
        pltpu.VMEM((N_EXPERTS, LANES), jnp.float32),
    ],
)(_sc_route)


@jax.jit
def kernel(x, W_expert, W_group):
    x2d = x.reshape(N_TOK, D_MODEL)
    w_cat = jnp.concatenate(
        [W_expert, W_group,
         jnp.zeros((W_ROWS - N_EXPERTS - N_GROUPS, D_MODEL), jnp.float32)],
        axis=0)
    parts = []
    for c in range(N_CHUNK):
        logits = _logits_tc(
            lax.slice_in_dim(x2d, c * TOK_C, (c + 1) * TOK_C, axis=0), w_cat)
        parts.append(_sc_route_call(logits))

    packed = jnp.concatenate(
        [jnp.transpose(p[0], (0, 2, 1)).reshape(TOK_C, N_EXPERTS + 2 * K)
         for p in parts],
        axis=0)
    probs_full = packed[:, :N_EXPERTS].reshape(4, 4096, N_EXPERTS)
    scores = packed[:, N_EXPERTS:N_EXPERTS + K].reshape(4, 4096, K)
    idx = packed[:, N_EXPERTS + K:].astype(jnp.int32).reshape(4, 4096, K)
    imp_p = sum(jnp.sum(p[1], axis=(0, 2)) for p in parts)
    cnt_p = sum(jnp.sum(p[2], axis=(0, 2)) for p in parts)
    importance = imp_p / float(N_TOK)
    load = cnt_p / jnp.maximum(jnp.sum(cnt_p), 1.0)
    return (idx, scores, probs_full, importance, load)
